# baseline (speedup 1.0000x reference)
import functools

import jax
import jax.numpy as jnp
from jax import lax
from jax.experimental import pallas as pl
from jax.experimental.pallas import tpu as pltpu
from jax.experimental.pallas import tpu_sc as plsc

B, N, D = 4, 2048, 256
R = B * N
NW = 32
ROWS_PER_W = R // NW
CAP = 1024
ECAP = NW * CAP
CHUNK = 64
TBLK = 512
ODIM = 16
PW = 128
TW = 2 * D + PW
GCAP = 1024


def _tc1_body(emb, ctrl, wft, wfb, w1t, w1b, bf, bd1, m1, m2, it_ref, jt_ref):
    e = emb[...]
    c = ctrl[...]
    f32 = jnp.float32
    it_ref[:, 0:D] = jnp.dot(e, wft[...], preferred_element_type=f32) + bf[...]
    it_ref[:, D:2 * D] = (jnp.dot(e, w1t[...], preferred_element_type=f32)
                          + bd1[...])
    it_ref[:, 2 * D:TW] = jnp.dot(c, m1[...], preferred_element_type=f32)
    jt_ref[:, 0:D] = jnp.dot(e, wfb[...], preferred_element_type=f32)
    jt_ref[:, D:2 * D] = jnp.dot(e, w1b[...], preferred_element_type=f32)
    jt_ref[:, 2 * D:TW] = jnp.dot(c, m2[...], preferred_element_type=f32)


def _tc1(emb2, ctrl2, wft, wfb, w1t, w1b, bf, bd1, m1, m2):
    nblk = R // TBLK
    row_spec = pl.BlockSpec((TBLK, D), lambda i: (i, 0))
    ctrl_spec = pl.BlockSpec((TBLK, 2 * 4), lambda i: (i, 0))
    t_spec = pl.BlockSpec((TBLK, TW), lambda i: (i, 0))
    full = lambda a: pl.BlockSpec(a.shape, lambda i: (0,) * a.ndim)
    return pl.pallas_call(
        _tc1_body,
        grid=(nblk,),
        in_specs=[row_spec, ctrl_spec, full(wft), full(wfb), full(w1t),
                  full(w1b), full(bf), full(bd1), full(m1), full(m2)],
        out_specs=[t_spec, t_spec],
        out_shape=[jax.ShapeDtypeStruct((R, TW), jnp.float32),
                   jax.ShapeDtypeStruct((R, TW), jnp.float32)],
    )(emb2, ctrl2, wft, wfb, w1t, w1b, bf, bd1, m1, m2)


def _tcb_body(conn, bm_ref):
    bm_ref[...] = jnp.sum(conn[...], axis=1)


def _tcb(conn128):
    blkg = 16384
    ng = R * 16
    return pl.pallas_call(
        _tcb_body,
        grid=(ng // blkg,),
        in_specs=[pl.BlockSpec((blkg, 128), lambda i: (i, 0))],
        out_specs=pl.BlockSpec((blkg,), lambda i: (i,)),
        out_shape=jax.ShapeDtypeStruct((ng,), jnp.float32),
    )(conn128)


def _sc1_body(conn128, bmap, itab, jtab,
              ids_out, counts_out, ie_out, je_out,
              brow, glist, g64, gbuf, edgebuf, stage16, ibuf, jbuf, ga, gb,
              cnt_ref, gcnt_ref, bsemA, gsemA, gsem0, gsem1, wsem0, wsem1):
    c = lax.axis_index("c")
    s = lax.axis_index("s")
    wid = c * 16 + s
    row0 = wid * ROWS_PER_W

    zero16 = jnp.zeros((16,), jnp.int32)

    def zb(t, carry):
        edgebuf[pl.ds(t * 16, 16)] = zero16
        glist[pl.ds(t * 16, 16)] = zero16
        return carry

    lax.fori_loop(0, (CAP + 16) // 16, zb, 0)
    cnt_ref[0] = 0
    gcnt_ref[0] = 0

    iota = lax.iota(jnp.int32, 16)

    pltpu.async_copy(bmap.at[pl.ds(row0 * 16, ROWS_PER_W * 16)], brow,
                     bsemA).wait()

    def rowA(r2, carry):
        fl = brow[pl.ds(r2 * 16, 16)] > 0.5
        gidv = jnp.full((16,), (row0 + r2) * 16, jnp.int32) + iota
        off = jnp.minimum(gcnt_ref[0], GCAP)
        plsc.store_compressed(glist.at[pl.ds(off, 16)], gidv, mask=fl)
        gcnt_ref[0] = off + plsc.all_reduce_population_count(fl)[0]
        return carry

    lax.fori_loop(0, ROWS_PER_W, rowA, 0)
    gcnt = gcnt_ref[0]

    ngch = lax.shift_right_logical(gcnt + 63, 6)

    def gch_body(k2, carry):
        for t in range(4):
            g64[pl.ds(t * 16, 16)] = glist[pl.ds(k2 * 64 + t * 16, 16)]
        pltpu.async_copy(conn128.at[g64], gbuf, gsemA).wait()
        ng = jnp.minimum(gcnt - k2 * 64, 64)

        def grp_body(u, carry2):
            gid = plsc.load_gather(g64, [jnp.full((16,), u, jnp.int32)])
            base = gid * 128
            for t in range(8):
                m = gbuf[u, pl.ds(t * 16, 16)] > 0.5
                npc = plsc.all_reduce_population_count(m)[0]
                idv = base + (t * 16) + iota
                off = jnp.minimum(cnt_ref[0], CAP)
                plsc.store_compressed(edgebuf.at[pl.ds(off, 16)], idv,
                                      mask=m)
                cnt_ref[0] = off + npc
            return carry2

        lax.fori_loop(0, ng, grp_body, 0)
        return carry

    lax.fori_loop(0, ngch, gch_body, 0)

    cnt = cnt_ref[0]
    stage16[...] = jnp.full((16,), cnt, jnp.int32)
    pltpu.sync_copy(stage16, counts_out.at[wid])
    pltpu.sync_copy(edgebuf.at[pl.ds(0, CAP)], ids_out.at[wid])

    nch = lax.shift_right_logical(cnt + (CHUNK - 1), 6)

    def ch_body(k, carry):
        @pl.when(k > 0)
        def _():
            pltpu.make_async_copy(ga, ie_out.at[pl.ds(0, CHUNK)],
                                  wsem0).wait()
            pltpu.make_async_copy(gb, je_out.at[pl.ds(0, CHUNK)],
                                  wsem1).wait()

        eoff = k * CHUNK
        for t in range(CHUNK // 16):
            idv = edgebuf[pl.ds(eoff + t * 16, 16)]
            rvec = lax.shift_right_logical(idv, 11)
            jv = jnp.bitwise_and(idv, N - 1)
            bvec = lax.shift_right_logical(idv, 22)
            jrow = jnp.bitwise_or(lax.shift_left(bvec, 11), jv)
            ibuf[pl.ds(t * 16, 16)] = rvec
            jbuf[pl.ds(t * 16, 16)] = jrow
        slot0 = wid * CAP + eoff
        hi = pltpu.async_copy(itab.at[ibuf], ga, gsem0)
        hj = pltpu.async_copy(jtab.at[jbuf], gb, gsem1)
        hi.wait()
        hj.wait()
        pltpu.async_copy(ga, ie_out.at[pl.ds(slot0, CHUNK)], wsem0)
        pltpu.async_copy(gb, je_out.at[pl.ds(slot0, CHUNK)], wsem1)
        return carry

    lax.fori_loop(0, nch, ch_body, 0)

    @pl.when(nch > 0)
    def _():
        pltpu.make_async_copy(ga, ie_out.at[pl.ds(0, CHUNK)], wsem0).wait()
        pltpu.make_async_copy(gb, je_out.at[pl.ds(0, CHUNK)], wsem1).wait()


def _sc1(conn128, bmap, itab, jtab):
    mesh = plsc.VectorSubcoreMesh(core_axis_name="c", subcore_axis_name="s")
    f = pl.kernel(
        _sc1_body,
        mesh=mesh,
        out_type=[
            jax.ShapeDtypeStruct((NW, CAP), jnp.int32),
            jax.ShapeDtypeStruct((NW, 16), jnp.int32),
            jax.ShapeDtypeStruct((ECAP, TW), jnp.float32),
            jax.ShapeDtypeStruct((ECAP, TW), jnp.float32),
        ],
        scratch_types=[
            pltpu.VMEM((ROWS_PER_W * 16,), jnp.float32),
            pltpu.VMEM((GCAP + 16,), jnp.int32),
            pltpu.VMEM((64,), jnp.int32),
            pltpu.VMEM((64, 128), jnp.float32),
            pltpu.VMEM((CAP + 16,), jnp.int32),
            pltpu.VMEM((16,), jnp.int32),
            pltpu.VMEM((CHUNK,), jnp.int32),
            pltpu.VMEM((CHUNK,), jnp.int32),
            pltpu.VMEM((CHUNK, TW), jnp.float32),
            pltpu.VMEM((CHUNK, TW), jnp.float32),
            pltpu.SMEM((1,), jnp.int32),
            pltpu.SMEM((1,), jnp.int32),
            pltpu.SemaphoreType.DMA,
            pltpu.SemaphoreType.DMA,
            pltpu.SemaphoreType.DMA,
            pltpu.SemaphoreType.DMA,
            pltpu.SemaphoreType.DMA,
            pltpu.SemaphoreType.DMA,
        ],
        compiler_params=pltpu.CompilerParams(needs_layout_passes=False),
    )
    return f(conn128, bmap, itab, jtab)


def _tc2_body(counts, ie, je, gam, bet, wd2, bd2, fused, loss, acc):
    i = pl.program_id(0)
    iv = ie[...]
    jv = je[...]
    pf = iv[:, 0:D] + jv[:, 0:D]
    mu = jnp.mean(pf, axis=1, keepdims=True)
    var = jnp.mean((pf - mu) ** 2, axis=1, keepdims=True)
    ln = (pf - mu) * lax.rsqrt(var + 1e-5) * gam[...] + bet[...]
    fus = jnp.maximum(ln, 0.0) * 0.1

    region = i // (CAP // TBLK)
    cnt = counts[region, 0]
    offs = lax.broadcasted_iota(jnp.int32, (TBLK, 1), 0) \
        + (i % (CAP // TBLK)) * TBLK
    within = offs < cnt
    fused[...] = jnp.where(within, fus, 0.0)

    h = jnp.maximum(iv[:, D:2 * D] + jv[:, D:2 * D], 0.0)
    pred = jnp.dot(h, wd2[...], preferred_element_type=jnp.float32) + bd2[...]
    pr = iv[:, 2 * D:2 * D + ODIM] + jv[:, 2 * D:2 * D + ODIM]
    l = jnp.sum(jnp.abs(pred - pr), axis=1, keepdims=True)
    bsum = jnp.sum(jnp.where(within, l, 0.0))

    @pl.when(i == 0)
    def _():
        acc[0] = 0.0

    acc[0] = acc[0] + bsum

    @pl.when(i == pl.num_programs(0) - 1)
    def _():
        def addc(r2, tot):
            return tot + counts[r2, 0]

        tote = lax.fori_loop(0, NW, addc, jnp.int32(0)).astype(jnp.float32)
        denom = jnp.maximum(tote * float(ODIM), 1.0)
        loss[0, 0] = jnp.where(tote > 0.5, acc[0] / denom, 0.0)


def _tc2(counts, ie, je, gam, bet, wd2, bd2):
    nblk = ECAP // TBLK
    t_spec = pl.BlockSpec((TBLK, TW), lambda i: (i, 0))
    full = lambda a: pl.BlockSpec(a.shape, lambda i: (0,) * a.ndim)
    return pl.pallas_call(
        _tc2_body,
        grid=(nblk,),
        in_specs=[pl.BlockSpec(memory_space=pltpu.SMEM),
                  t_spec, t_spec, full(gam), full(bet), full(wd2), full(bd2)],
        out_specs=[pl.BlockSpec((TBLK, D), lambda i: (i, 0)),
                   pl.BlockSpec((1, 1), lambda i: (0, 0),
                                memory_space=pltpu.SMEM)],
        out_shape=[jax.ShapeDtypeStruct((ECAP, D), jnp.float32),
                   jax.ShapeDtypeStruct((1, 1), jnp.float32)],
        scratch_shapes=[pltpu.SMEM((1,), jnp.float32)],
    )(counts, ie, je, gam, bet, wd2, bd2)


MCAP = 2048


def _sc2_body(emb, fusedv, ids, counts, out,
              accl, idall, cvall, mslot, mdest, slot64, gfb, mcnt_ref):
    c = lax.axis_index("c")
    s = lax.axis_index("s")
    wid = c * 16 + s

    pltpu.sync_copy(emb.at[pl.ds(wid * ROWS_PER_W, ROWS_PER_W)], accl)
    pltpu.sync_copy(ids, idall)
    pltpu.sync_copy(counts, cvall)

    zero16 = jnp.zeros((16,), jnp.int32)

    def zb(t, carry):
        mslot[pl.ds(t * 16, 16)] = zero16
        mdest[pl.ds(t * 16, 16)] = zero16
        return carry

    lax.fori_loop(0, (MCAP + 16) // 16, zb, 0)
    mcnt_ref[0] = 0

    iota = lax.iota(jnp.int32, 16)

    def reg_body(w2, carry):
        cntw = cvall[w2, pl.ds(0, 16)][0]
        ngr = lax.shift_right_logical(cntw + 63, 6)

        def grp_body(g, carry2):
            tests = []
            for t in range(4):
                p0 = g * 64 + t * 16
                idv = idall[w2, pl.ds(p0, 16)]
                pv = jnp.full((16,), p0, jnp.int32) + iota
                valid = pv < cntw
                irow = lax.shift_right_logical(idv, 11)
                jrow = jnp.bitwise_or(
                    lax.shift_left(lax.shift_right_logical(idv, 22), 11),
                    jnp.bitwise_and(idv, N - 1))
                mi = jnp.logical_and(valid,
                                     lax.shift_right_logical(irow, 8) == wid)
                mj = jnp.logical_and(valid,
                                     lax.shift_right_logical(jrow, 8) == wid)
                tests.append((p0, irow, jrow, mi, mj))
            o01 = jnp.logical_or(
                jnp.logical_or(tests[0][3], tests[0][4]),
                jnp.logical_or(tests[1][3], tests[1][4]))
            o23 = jnp.logical_or(
                jnp.logical_or(tests[2][3], tests[2][4]),
                jnp.logical_or(tests[3][3], tests[3][4]))
            anyv = plsc.all_reduce_population_count(
                jnp.logical_or(o01, o23))[0]

            @pl.when(anyv > 0)
            def _():
                for p0, irow, jrow, mi, mj in tests:
                    pci = plsc.all_reduce_population_count(mi)[0]
                    pcj = plsc.all_reduce_population_count(mj)[0]

                    @pl.when(pci + pcj > 0)
                    def _():
                        slotv = jnp.full((16,), w2 * CAP + p0,
                                         jnp.int32) + iota
                        off = jnp.minimum(mcnt_ref[0], MCAP)
                        plsc.store_compressed(mslot.at[pl.ds(off, 16)],
                                              slotv, mask=mi)
                        plsc.store_compressed(
                            mdest.at[pl.ds(off, 16)],
                            jnp.bitwise_and(irow, ROWS_PER_W - 1), mask=mi)
                        off = jnp.minimum(off + pci, MCAP)
                        plsc.store_compressed(mslot.at[pl.ds(off, 16)],
                                              slotv, mask=mj)
                        plsc.store_compressed(
                            mdest.at[pl.ds(off, 16)],
                            jnp.bitwise_and(jrow, ROWS_PER_W - 1), mask=mj)
                        mcnt_ref[0] = off + pcj

            return carry2

        lax.fori_loop(0, ngr, grp_body, 0)
        return carry

    lax.fori_loop(0, NW, reg_body, 0)

    cols = [iota + t * 16 for t in range(D // 16)]
    mcnt = mcnt_ref[0]
    nch = lax.shift_right_logical(mcnt + (CHUNK - 1), 6)

    def ch_body(k, carry):
        for t in range(CHUNK // 16):
            slot64[pl.ds(t * 16, 16)] = mslot[pl.ds(k * CHUNK + t * 16, 16)]
        pltpu.sync_copy(fusedv.at[slot64], gfb)
        ne = jnp.minimum(mcnt - k * CHUNK, CHUNK)
        nq = lax.shift_right_logical(ne + 3, 2)

        def ed_body(q, carry2):
            for u in range(4):
                e = q * 4 + u
                me = k * CHUNK + e
                mev = jnp.full((16,), me, jnp.int32)
                maskv = mev < mcnt
                esplat = jnp.full((16,), e, jnp.int32)
                rowv = plsc.load_gather(mdest, [mev])
                for t in range(D // 16):
                    v = plsc.load_gather(gfb, [esplat, cols[t]])
                    plsc.addupdate_scatter(accl, [rowv, cols[t]], v,
                                           mask=maskv)
            return carry2

        lax.fori_loop(0, nq, ed_body, 0)
        return carry

    lax.fori_loop(0, nch, ch_body, 0)

    pltpu.sync_copy(accl, out.at[pl.ds(wid * ROWS_PER_W, ROWS_PER_W)])


def _sc2(emb2, fusedv, ids, counts):
    mesh = plsc.VectorSubcoreMesh(core_axis_name="c", subcore_axis_name="s")
    f = pl.kernel(
        _sc2_body,
        mesh=mesh,
        out_type=jax.ShapeDtypeStruct((R, D), jnp.float32),
        scratch_types=[
            pltpu.VMEM((ROWS_PER_W, D), jnp.float32),
            pltpu.VMEM((NW, CAP), jnp.int32),
            pltpu.VMEM((NW, 16), jnp.int32),
            pltpu.VMEM((MCAP + 16,), jnp.int32),
            pltpu.VMEM((MCAP + 16,), jnp.int32),
            pltpu.VMEM((CHUNK,), jnp.int32),
            pltpu.VMEM((CHUNK, D), jnp.float32),
            pltpu.SMEM((1,), jnp.int32),
        ],
        compiler_params=pltpu.CompilerParams(needs_layout_passes=False),
    )
    return f(emb2, fusedv, ids, counts)


def kernel(lane_embeddings, lane_ctrl_points, connectivity, Wf, bf, gamma,
           beta, Wd1, bd1, Wd2, bd2, bezier_matrix):
    emb2 = lane_embeddings.reshape(R, D)
    ctrl2 = lane_ctrl_points.reshape(R, 8)

    wft, wfb = Wf[:D], Wf[D:]
    w1t, w1b = Wd1[:D], Wd1[D:]
    eye2 = jnp.eye(2, dtype=jnp.float32)
    m1 = jnp.pad(jnp.kron(bezier_matrix[:, :4].T, eye2),
                 ((0, 0), (0, PW - ODIM)))
    m2 = jnp.pad(jnp.kron(bezier_matrix[:, 4:].T, eye2),
                 ((0, 0), (0, PW - ODIM)))

    conn128 = connectivity.reshape(R * 16, 128)
    bmap = _tcb(conn128)
    itab, jtab = _tc1(emb2, ctrl2, wft, wfb, w1t, w1b,
                      bf.reshape(1, D), bd1.reshape(1, D), m1, m2)

    ids, counts, ie, je = _sc1(conn128, bmap, itab, jtab)

    fusedv, loss = _tc2(counts, ie, je,
                        gamma.reshape(1, D), beta.reshape(1, D),
                        Wd2, bd2.reshape(1, ODIM))

    enhanced2 = _sc2(emb2, fusedv, ids, counts)

    return loss[0, 0], enhanced2.reshape(B, N, D)


# speedup vs baseline: 1.1131x; 1.1131x over previous
import functools

import jax
import jax.numpy as jnp
from jax import lax
from jax.experimental import pallas as pl
from jax.experimental.pallas import tpu as pltpu
from jax.experimental.pallas import tpu_sc as plsc

B, N, D = 4, 2048, 256
R = B * N
NW = 32
ROWS_PER_W = R // NW
CAP = 1024
ECAP = NW * CAP
CHUNK = 64
TBLK = 512
ODIM = 16
PW = 128
TW = 2 * D + PW
NB = 8


def _tc1_body(emb, ctrl, wft, wfb, w1t, w1b, bf, bd1, m1, m2, it_ref, jt_ref):
    e = emb[...]
    c = ctrl[...]
    f32 = jnp.float32
    it_ref[:, 0:D] = jnp.dot(e, wft[...], preferred_element_type=f32) + bf[...]
    it_ref[:, D:2 * D] = (jnp.dot(e, w1t[...], preferred_element_type=f32)
                          + bd1[...])
    it_ref[:, 2 * D:TW] = jnp.dot(c, m1[...], preferred_element_type=f32)
    jt_ref[:, 0:D] = jnp.dot(e, wfb[...], preferred_element_type=f32)
    jt_ref[:, D:2 * D] = jnp.dot(e, w1b[...], preferred_element_type=f32)
    jt_ref[:, 2 * D:TW] = jnp.dot(c, m2[...], preferred_element_type=f32)


def _tc1(emb2, ctrl2, wft, wfb, w1t, w1b, bf, bd1, m1, m2):
    nblk = R // TBLK
    row_spec = pl.BlockSpec((TBLK, D), lambda i: (i, 0))
    ctrl_spec = pl.BlockSpec((TBLK, 2 * 4), lambda i: (i, 0))
    t_spec = pl.BlockSpec((TBLK, TW), lambda i: (i, 0))
    full = lambda a: pl.BlockSpec(a.shape, lambda i: (0,) * a.ndim)
    return pl.pallas_call(
        _tc1_body,
        grid=(nblk,),
        in_specs=[row_spec, ctrl_spec, full(wft), full(wfb), full(w1t),
                  full(w1b), full(bf), full(bd1), full(m1), full(m2)],
        out_specs=[t_spec, t_spec],
        out_shape=[jax.ShapeDtypeStruct((R, TW), jnp.float32),
                   jax.ShapeDtypeStruct((R, TW), jnp.float32)],
    )(emb2, ctrl2, wft, wfb, w1t, w1b, bf, bd1, m1, m2)


def _sc1_body(conn, itab, jtab,
              ids_out, counts_out, ie_out, je_out,
              rowbuf, edgebuf, stage16, ibuf, jbuf, ga, gb,
              cnt_ref, csem0, csem1, gsem0, gsem1, wsem0, wsem1):
    c = lax.axis_index("c")
    s = lax.axis_index("s")
    wid = c * 16 + s
    row0 = wid * ROWS_PER_W

    zero16 = jnp.zeros((16,), jnp.int32)

    def zb(t, carry):
        edgebuf[pl.ds(t * 16, 16)] = zero16
        return carry

    lax.fori_loop(0, (CAP + 16) // 16, zb, 0)
    cnt_ref[0] = 0

    iota = lax.iota(jnp.int32, 16)
    nbatch = ROWS_PER_W // NB
    csems = (csem0, csem1)

    def scan_batch(b, slot):
        def row_body(r2, carry):
            idrow = (row0 + b * NB) * N + r2 * N

            def grp_body(g, carry2):
                base = g * 128
                ms = [rowbuf[slot, r2, pl.ds(base + t * 16, 16)] > 0.5
                      for t in range(8)]
                o0 = jnp.logical_or(jnp.logical_or(ms[0], ms[1]),
                                    jnp.logical_or(ms[2], ms[3]))
                o1 = jnp.logical_or(jnp.logical_or(ms[4], ms[5]),
                                    jnp.logical_or(ms[6], ms[7]))
                anyg = plsc.all_reduce_population_count(
                    jnp.logical_or(o0, o1))[0]

                @pl.when(anyg > 0)
                def _():
                    for t in range(8):
                        npc = plsc.all_reduce_population_count(ms[t])[0]

                        @pl.when(npc > 0)
                        def _():
                            idv = jnp.full((16,), idrow + base + t * 16,
                                           jnp.int32) + iota
                            off = jnp.minimum(cnt_ref[0], CAP)
                            plsc.store_compressed(
                                edgebuf.at[pl.ds(off, 16)], idv, mask=ms[t])
                            cnt_ref[0] = off + npc

                return carry2

            lax.fori_loop(0, 16, grp_body, 0)
            return carry

        lax.fori_loop(0, NB, row_body, 0)

    handles = [None, None]
    handles[0] = pltpu.async_copy(conn.at[pl.ds(row0, NB)], rowbuf.at[0],
                                  csems[0])
    for b in range(nbatch):
        slot = b % 2
        handles[slot].wait()
        if b + 1 < nbatch:
            handles[1 - slot] = pltpu.async_copy(
                conn.at[pl.ds(row0 + (b + 1) * NB, NB)],
                rowbuf.at[1 - slot], csems[1 - slot])
        scan_batch(b, slot)

    cnt = cnt_ref[0]
    stage16[...] = jnp.full((16,), cnt, jnp.int32)
    pltpu.sync_copy(stage16, counts_out.at[wid])
    pltpu.sync_copy(edgebuf.at[pl.ds(0, CAP)], ids_out.at[wid])

    nch = lax.shift_right_logical(cnt + (CHUNK - 1), 6)

    def ch_body(k, carry):
        @pl.when(k > 0)
        def _():
            pltpu.make_async_copy(ga, ie_out.at[pl.ds(0, CHUNK)],
                                  wsem0).wait()
            pltpu.make_async_copy(gb, je_out.at[pl.ds(0, CHUNK)],
                                  wsem1).wait()

        eoff = k * CHUNK
        for t in range(CHUNK // 16):
            idv = edgebuf[pl.ds(eoff + t * 16, 16)]
            rvec = lax.shift_right_logical(idv, 11)
            jv = jnp.bitwise_and(idv, N - 1)
            bvec = lax.shift_right_logical(idv, 22)
            jrow = jnp.bitwise_or(lax.shift_left(bvec, 11), jv)
            ibuf[pl.ds(t * 16, 16)] = rvec
            jbuf[pl.ds(t * 16, 16)] = jrow
        slot0 = wid * CAP + eoff
        hi = pltpu.async_copy(itab.at[ibuf], ga, gsem0)
        hj = pltpu.async_copy(jtab.at[jbuf], gb, gsem1)
        hi.wait()
        hj.wait()
        pltpu.async_copy(ga, ie_out.at[pl.ds(slot0, CHUNK)], wsem0)
        pltpu.async_copy(gb, je_out.at[pl.ds(slot0, CHUNK)], wsem1)
        return carry

    lax.fori_loop(0, nch, ch_body, 0)

    @pl.when(nch > 0)
    def _():
        pltpu.make_async_copy(ga, ie_out.at[pl.ds(0, CHUNK)], wsem0).wait()
        pltpu.make_async_copy(gb, je_out.at[pl.ds(0, CHUNK)], wsem1).wait()


def _sc1(conn2, itab, jtab):
    mesh = plsc.VectorSubcoreMesh(core_axis_name="c", subcore_axis_name="s")
    f = pl.kernel(
        _sc1_body,
        mesh=mesh,
        out_type=[
            jax.ShapeDtypeStruct((NW, CAP), jnp.int32),
            jax.ShapeDtypeStruct((NW, 16), jnp.int32),
            jax.ShapeDtypeStruct((ECAP, TW), jnp.float32),
            jax.ShapeDtypeStruct((ECAP, TW), jnp.float32),
        ],
        scratch_types=[
            pltpu.VMEM((2, NB, N), jnp.float32),
            pltpu.VMEM((CAP + 16,), jnp.int32),
            pltpu.VMEM((16,), jnp.int32),
            pltpu.VMEM((CHUNK,), jnp.int32),
            pltpu.VMEM((CHUNK,), jnp.int32),
            pltpu.VMEM((CHUNK, TW), jnp.float32),
            pltpu.VMEM((CHUNK, TW), jnp.float32),
            pltpu.SMEM((1,), jnp.int32),
            pltpu.SemaphoreType.DMA,
            pltpu.SemaphoreType.DMA,
            pltpu.SemaphoreType.DMA,
            pltpu.SemaphoreType.DMA,
            pltpu.SemaphoreType.DMA,
            pltpu.SemaphoreType.DMA,
        ],
        compiler_params=pltpu.CompilerParams(needs_layout_passes=False),
    )
    return f(conn2, itab, jtab)


def _tc2_body(counts, ie, je, gam, bet, wd2, bd2, fused, loss, acc):
    i = pl.program_id(0)
    iv = ie[...]
    jv = je[...]
    pf = iv[:, 0:D] + jv[:, 0:D]
    mu = jnp.mean(pf, axis=1, keepdims=True)
    var = jnp.mean((pf - mu) ** 2, axis=1, keepdims=True)
    ln = (pf - mu) * lax.rsqrt(var + 1e-5) * gam[...] + bet[...]
    fus = jnp.maximum(ln, 0.0) * 0.1

    region = i // (CAP // TBLK)
    cnt = counts[region, 0]
    offs = lax.broadcasted_iota(jnp.int32, (TBLK, 1), 0) \
        + (i % (CAP // TBLK)) * TBLK
    within = offs < cnt
    fused[...] = jnp.where(within, fus, 0.0)

    h = jnp.maximum(iv[:, D:2 * D] + jv[:, D:2 * D], 0.0)
    pred = jnp.dot(h, wd2[...], preferred_element_type=jnp.float32) + bd2[...]
    pr = iv[:, 2 * D:2 * D + ODIM] + jv[:, 2 * D:2 * D + ODIM]
    l = jnp.sum(jnp.abs(pred - pr), axis=1, keepdims=True)
    bsum = jnp.sum(jnp.where(within, l, 0.0))

    @pl.when(i == 0)
    def _():
        acc[0] = 0.0

    acc[0] = acc[0] + bsum

    @pl.when(i == pl.num_programs(0) - 1)
    def _():
        def addc(r2, tot):
            return tot + counts[r2, 0]

        tote = lax.fori_loop(0, NW, addc, jnp.int32(0)).astype(jnp.float32)
        denom = jnp.maximum(tote * float(ODIM), 1.0)
        loss[0, 0] = jnp.where(tote > 0.5, acc[0] / denom, 0.0)


def _tc2(counts, ie, je, gam, bet, wd2, bd2):
    nblk = ECAP // TBLK
    t_spec = pl.BlockSpec((TBLK, TW), lambda i: (i, 0))
    full = lambda a: pl.BlockSpec(a.shape, lambda i: (0,) * a.ndim)
    return pl.pallas_call(
        _tc2_body,
        grid=(nblk,),
        in_specs=[pl.BlockSpec(memory_space=pltpu.SMEM),
                  t_spec, t_spec, full(gam), full(bet), full(wd2), full(bd2)],
        out_specs=[pl.BlockSpec((TBLK, D), lambda i: (i, 0)),
                   pl.BlockSpec((1, 1), lambda i: (0, 0),
                                memory_space=pltpu.SMEM)],
        out_shape=[jax.ShapeDtypeStruct((ECAP, D), jnp.float32),
                   jax.ShapeDtypeStruct((1, 1), jnp.float32)],
        scratch_shapes=[pltpu.SMEM((1,), jnp.float32)],
    )(counts, ie, je, gam, bet, wd2, bd2)


MCAP = 2048


def _sc2_body(emb, fusedv, ids, counts, out,
              accl, idall, cvall, mslot, mdest, slot64, gfb, mcnt_ref):
    c = lax.axis_index("c")
    s = lax.axis_index("s")
    wid = c * 16 + s

    pltpu.sync_copy(emb.at[pl.ds(wid * ROWS_PER_W, ROWS_PER_W)], accl)
    pltpu.sync_copy(ids, idall)
    pltpu.sync_copy(counts, cvall)

    zero16 = jnp.zeros((16,), jnp.int32)

    def zb(t, carry):
        mslot[pl.ds(t * 16, 16)] = zero16
        mdest[pl.ds(t * 16, 16)] = zero16
        return carry

    lax.fori_loop(0, (MCAP + 16) // 16, zb, 0)
    mcnt_ref[0] = 0

    iota = lax.iota(jnp.int32, 16)

    def reg_body(w2, carry):
        cntw = cvall[w2, pl.ds(0, 16)][0]
        ngr = lax.shift_right_logical(cntw + 63, 6)

        def grp_body(g, carry2):
            tests = []
            for t in range(4):
                p0 = g * 64 + t * 16
                idv = idall[w2, pl.ds(p0, 16)]
                pv = jnp.full((16,), p0, jnp.int32) + iota
                valid = pv < cntw
                irow = lax.shift_right_logical(idv, 11)
                jrow = jnp.bitwise_or(
                    lax.shift_left(lax.shift_right_logical(idv, 22), 11),
                    jnp.bitwise_and(idv, N - 1))
                mi = jnp.logical_and(valid,
                                     lax.shift_right_logical(irow, 8) == wid)
                mj = jnp.logical_and(valid,
                                     lax.shift_right_logical(jrow, 8) == wid)
                tests.append((p0, irow, jrow, mi, mj))
            o01 = jnp.logical_or(
                jnp.logical_or(tests[0][3], tests[0][4]),
                jnp.logical_or(tests[1][3], tests[1][4]))
            o23 = jnp.logical_or(
                jnp.logical_or(tests[2][3], tests[2][4]),
                jnp.logical_or(tests[3][3], tests[3][4]))
            anyv = plsc.all_reduce_population_count(
                jnp.logical_or(o01, o23))[0]

            @pl.when(anyv > 0)
            def _():
                for p0, irow, jrow, mi, mj in tests:
                    pci = plsc.all_reduce_population_count(mi)[0]
                    pcj = plsc.all_reduce_population_count(mj)[0]

                    @pl.when(pci + pcj > 0)
                    def _():
                        slotv = jnp.full((16,), w2 * CAP + p0,
                                         jnp.int32) + iota
                        off = jnp.minimum(mcnt_ref[0], MCAP)
                        plsc.store_compressed(mslot.at[pl.ds(off, 16)],
                                              slotv, mask=mi)
                        plsc.store_compressed(
                            mdest.at[pl.ds(off, 16)],
                            jnp.bitwise_and(irow, ROWS_PER_W - 1), mask=mi)
                        off = jnp.minimum(off + pci, MCAP)
                        plsc.store_compressed(mslot.at[pl.ds(off, 16)],
                                              slotv, mask=mj)
                        plsc.store_compressed(
                            mdest.at[pl.ds(off, 16)],
                            jnp.bitwise_and(jrow, ROWS_PER_W - 1), mask=mj)
                        mcnt_ref[0] = off + pcj

            return carry2

        lax.fori_loop(0, ngr, grp_body, 0)
        return carry

    lax.fori_loop(0, NW, reg_body, 0)

    cols = [iota + t * 16 for t in range(D // 16)]
    mcnt = mcnt_ref[0]
    nch = lax.shift_right_logical(mcnt + (CHUNK - 1), 6)

    def ch_body(k, carry):
        for t in range(CHUNK // 16):
            slot64[pl.ds(t * 16, 16)] = mslot[pl.ds(k * CHUNK + t * 16, 16)]
        pltpu.sync_copy(fusedv.at[slot64], gfb)
        ne = jnp.minimum(mcnt - k * CHUNK, CHUNK)
        nq = lax.shift_right_logical(ne + 3, 2)

        def ed_body(q, carry2):
            for u in range(4):
                e = q * 4 + u
                me = k * CHUNK + e
                mev = jnp.full((16,), me, jnp.int32)
                maskv = mev < mcnt
                esplat = jnp.full((16,), e, jnp.int32)
                rowv = plsc.load_gather(mdest, [mev])
                for t in range(D // 16):
                    v = plsc.load_gather(gfb, [esplat, cols[t]])
                    plsc.addupdate_scatter(accl, [rowv, cols[t]], v,
                                           mask=maskv)
            return carry2

        lax.fori_loop(0, nq, ed_body, 0)
        return carry

    lax.fori_loop(0, nch, ch_body, 0)

    pltpu.sync_copy(accl, out.at[pl.ds(wid * ROWS_PER_W, ROWS_PER_W)])


def _sc2(emb2, fusedv, ids, counts):
    mesh = plsc.VectorSubcoreMesh(core_axis_name="c", subcore_axis_name="s")
    f = pl.kernel(
        _sc2_body,
        mesh=mesh,
        out_type=jax.ShapeDtypeStruct((R, D), jnp.float32),
        scratch_types=[
            pltpu.VMEM((ROWS_PER_W, D), jnp.float32),
            pltpu.VMEM((NW, CAP), jnp.int32),
            pltpu.VMEM((NW, 16), jnp.int32),
            pltpu.VMEM((MCAP + 16,), jnp.int32),
            pltpu.VMEM((MCAP + 16,), jnp.int32),
            pltpu.VMEM((CHUNK,), jnp.int32),
            pltpu.VMEM((CHUNK, D), jnp.float32),
            pltpu.SMEM((1,), jnp.int32),
        ],
        compiler_params=pltpu.CompilerParams(needs_layout_passes=False),
    )
    return f(emb2, fusedv, ids, counts)


def kernel(lane_embeddings, lane_ctrl_points, connectivity, Wf, bf, gamma,
           beta, Wd1, bd1, Wd2, bd2, bezier_matrix):
    emb2 = lane_embeddings.reshape(R, D)
    conn2 = connectivity.reshape(R, N)
    ctrl2 = lane_ctrl_points.reshape(R, 8)

    wft, wfb = Wf[:D], Wf[D:]
    w1t, w1b = Wd1[:D], Wd1[D:]
    eye2 = jnp.eye(2, dtype=jnp.float32)
    m1 = jnp.pad(jnp.kron(bezier_matrix[:, :4].T, eye2),
                 ((0, 0), (0, PW - ODIM)))
    m2 = jnp.pad(jnp.kron(bezier_matrix[:, 4:].T, eye2),
                 ((0, 0), (0, PW - ODIM)))

    itab, jtab = _tc1(emb2, ctrl2, wft, wfb, w1t, w1b,
                      bf.reshape(1, D), bd1.reshape(1, D), m1, m2)

    ids, counts, ie, je = _sc1(conn2, itab, jtab)

    fusedv, loss = _tc2(counts, ie, je,
                        gamma.reshape(1, D), beta.reshape(1, D),
                        Wd2, bd2.reshape(1, ODIM))

    enhanced2 = _sc2(emb2, fusedv, ids, counts)

    return loss[0, 0], enhanced2.reshape(B, N, D)
